# baseline (speedup 1.0000x reference)
import functools

import jax
import jax.numpy as jnp
from jax import lax
from jax.experimental import pallas as pl
from jax.experimental.pallas import tpu as pltpu
from jax.experimental.pallas import tpu_sc as plsc

NC, NS = 2, 16
NW = NC * NS
D = 128
PP = 20
NN = 200
R = 224
RH = R // 2


def _sc_dots(combined_idx, input_labels, in_embed, out_embed, B):
    b_per_w = B // NW
    mesh = plsc.VectorSubcoreMesh(core_axis_name="c", subcore_axis_name="s")

    @functools.partial(
        pl.kernel,
        mesh=mesh,
        compiler_params=pltpu.CompilerParams(needs_layout_passes=False),
        out_type=jax.ShapeDtypeStruct((B, R), jnp.float32),
        scratch_types=[
            pltpu.VMEM((b_per_w,), jnp.int32),
            pltpu.VMEM((b_per_w, D), jnp.float32),
            pltpu.VMEM((b_per_w * R,), jnp.int32),
            pltpu.VMEM((2 * R, D), jnp.float32),
            pltpu.VMEM((R,), jnp.float32),
            pltpu.VMEM((R,), jnp.float32),
            pltpu.SemaphoreType.DMA,
            pltpu.SemaphoreType.DMA,
            pltpu.SemaphoreType.DMA,
            pltpu.SemaphoreType.DMA,
        ],
    )
    def k(idx_hbm, cidx_hbm, in_tab, out_tab, out_hbm,
          cidx_v, cent_v, idx_v, rows_v, dots0_v, dots1_v, s0, s1, d0, d1):
        wid = lax.axis_index("s") * NC + lax.axis_index("c")
        base = wid * b_per_w
        pltpu.sync_copy(cidx_hbm.at[pl.ds(base, b_per_w)], cidx_v)
        pltpu.sync_copy(idx_hbm.at[pl.ds(base * R, b_per_w * R)], idx_v)
        pltpu.async_copy(in_tab.at[cidx_v], cent_v, s0).wait()
        lane = lax.iota(jnp.int32, 16)

        def fire(b, row_off, sem):
            off = pl.multiple_of(b * R, 8)
            pltpu.async_copy(
                out_tab.at[idx_v.at[pl.ds(off, RH)]],
                rows_v.at[pl.ds(row_off, RH)], sem)
            pltpu.async_copy(
                out_tab.at[idx_v.at[pl.ds(off + RH, RH)]],
                rows_v.at[pl.ds(row_off + RH, RH)], sem)

        def wait_rows(sem, row_off):
            pltpu.make_async_copy(
                out_tab.at[pl.ds(0, R)],
                rows_v.at[pl.ds(row_off, R)], sem).wait()

        def wait_dots(dots_ref, sem):
            pltpu.make_async_copy(
                dots_ref, out_hbm.at[base], sem).wait()

        def compute(row_base, bl, dots_ref):
            cs = [cent_v[bl, pl.ds(16 * j, 16)] for j in range(8)]

            def per_g(g, c2):
                rows16 = row_base + g * 16 + lane
                accs = []
                for j in range(8):
                    acc = jnp.zeros((16,), jnp.float32)
                    for l in range(16):
                        col = jnp.full((16,), 16 * j + l, jnp.int32)
                        v = plsc.load_gather(rows_v, [rows16, col])
                        acc = acc + v * cs[j][l]
                    accs.append(acc)
                tot = (((accs[0] + accs[1]) + (accs[2] + accs[3]))
                       + ((accs[4] + accs[5]) + (accs[6] + accs[7])))
                dots_ref[pl.ds(pl.multiple_of(g * 16, 16), 16)] = tot
                return c2

            lax.fori_loop(0, R // 16, per_g, 0)

        fire(0, 0, s0)
        fire(1, R, s1)
        nt = b_per_w // 2

        def body(t, carry):
            b0 = 2 * t
            wait_rows(s0, 0)

            @pl.when(t > 0)
            def _():
                wait_dots(dots0_v, d0)

            compute(0, b0, dots0_v)

            @pl.when(t < nt - 1)
            def _():
                fire(b0 + 2, 0, s0)

            pltpu.async_copy(dots0_v, out_hbm.at[base + b0], d0)

            wait_rows(s1, R)

            @pl.when(t > 0)
            def _():
                wait_dots(dots1_v, d1)

            compute(R, b0 + 1, dots1_v)

            @pl.when(t < nt - 1)
            def _():
                fire(b0 + 3, R, s1)

            pltpu.async_copy(dots1_v, out_hbm.at[base + b0 + 1], d1)
            return carry

        lax.fori_loop(0, nt, body, 0)
        wait_dots(dots0_v, d0)
        wait_dots(dots1_v, d1)

    return k(combined_idx, input_labels, in_embed, out_embed)


def _tc_loss(dots, B):
    bblk = 512

    def body(d_ref, o_ref):
        x = d_ref[...]
        col = lax.broadcasted_iota(jnp.int32, x.shape, 1)
        y = jnp.where(col < PP, x, -x)
        ls = jax.nn.log_sigmoid(y)
        ls = jnp.where(col < PP + NN, ls, 0.0)
        o_ref[...] = -jnp.sum(ls, axis=1)

    return pl.pallas_call(
        body,
        grid=(B // bblk,),
        in_specs=[pl.BlockSpec((bblk, R), lambda i: (i, 0))],
        out_specs=pl.BlockSpec((bblk,), lambda i: (i,)),
        out_shape=jax.ShapeDtypeStruct((B,), jnp.float32),
    )(dots)


def kernel(input_labels, pos_labels, neg_labels, in_embed, out_embed):
    B = input_labels.shape[0]
    pad = jnp.zeros((B, R - PP - NN), jnp.int32)
    combined = jnp.concatenate(
        [pos_labels, neg_labels, pad], axis=1).reshape(-1)
    dots = _sc_dots(combined, input_labels, in_embed, out_embed, B)
    return _tc_loss(dots, B)


# speedup vs baseline: 2.2915x; 2.2915x over previous
import functools

import jax
import jax.numpy as jnp
from jax import lax
from jax.experimental import pallas as pl
from jax.experimental.pallas import tpu as pltpu
from jax.experimental.pallas import tpu_sc as plsc

NC, NS = 2, 16
NW = NC * NS
D = 128
PP = 20
NN = 200
R = 224
RH = R // 2


def _sc_dots(combined_idx, input_labels, in_embed, out_embed, B):
    b_per_w = B // NW
    mesh = plsc.VectorSubcoreMesh(core_axis_name="c", subcore_axis_name="s")

    @functools.partial(
        pl.kernel,
        mesh=mesh,
        compiler_params=pltpu.CompilerParams(needs_layout_passes=False),
        out_type=jax.ShapeDtypeStruct((B, R), jnp.float32),
        scratch_types=[
            pltpu.VMEM((b_per_w,), jnp.int32),
            pltpu.VMEM((b_per_w, D), jnp.float32),
            pltpu.VMEM((b_per_w * R,), jnp.int32),
            pltpu.VMEM((2 * R, D), jnp.float32),
            pltpu.VMEM((R,), jnp.float32),
            pltpu.VMEM((R,), jnp.float32),
            pltpu.SemaphoreType.DMA,
            pltpu.SemaphoreType.DMA,
            pltpu.SemaphoreType.DMA,
            pltpu.SemaphoreType.DMA,
        ],
    )
    def k(idx_hbm, cidx_hbm, in_tab, out_tab, out_hbm,
          cidx_v, cent_v, idx_v, rows_v, dots0_v, dots1_v, s0, s1, d0, d1):
        wid = lax.axis_index("s") * NC + lax.axis_index("c")
        base = wid * b_per_w
        pltpu.sync_copy(cidx_hbm.at[pl.ds(base, b_per_w)], cidx_v)
        pltpu.sync_copy(idx_hbm.at[pl.ds(base * R, b_per_w * R)], idx_v)
        pltpu.async_copy(in_tab.at[cidx_v], cent_v, s0).wait()
        lane = lax.iota(jnp.int32, 16)

        def fire(b, row_off, sem):
            off = pl.multiple_of(b * R, 8)
            pltpu.async_copy(
                out_tab.at[idx_v.at[pl.ds(off, RH)]],
                rows_v.at[pl.ds(row_off, RH)], sem)
            pltpu.async_copy(
                out_tab.at[idx_v.at[pl.ds(off + RH, RH)]],
                rows_v.at[pl.ds(row_off + RH, RH)], sem)

        def wait_rows(sem, row_off):
            pltpu.make_async_copy(
                out_tab.at[pl.ds(0, R)],
                rows_v.at[pl.ds(row_off, R)], sem).wait()

        def wait_dots(dots_ref, sem):
            pltpu.make_async_copy(
                dots_ref, out_hbm.at[base], sem).wait()

        m15 = lane == 15

        def compute(row_base, bl, dots_ref):
            cs = [cent_v[bl, pl.ds(16 * j, 16)] for j in range(8)]

            def per8(g, c2):
                for k in range(8):
                    r = g * 8 + k
                    row = row_base + r
                    acc = rows_v[row, pl.ds(0, 16)] * cs[0]
                    for j in range(1, 8):
                        acc = acc + rows_v[row, pl.ds(16 * j, 16)] * cs[j]
                    s = plsc.cumsum(acc)
                    plsc.store_scatter(
                        dots_ref, [jnp.full((16,), r, jnp.int32)], s,
                        mask=m15)
                return c2

            lax.fori_loop(0, R // 8, per8, 0)

        fire(0, 0, s0)
        fire(1, R, s1)
        nt = b_per_w // 2

        def body(t, carry):
            b0 = 2 * t
            wait_rows(s0, 0)

            @pl.when(t > 0)
            def _():
                wait_dots(dots0_v, d0)

            compute(0, b0, dots0_v)

            @pl.when(t < nt - 1)
            def _():
                fire(b0 + 2, 0, s0)

            pltpu.async_copy(dots0_v, out_hbm.at[base + b0], d0)

            wait_rows(s1, R)

            @pl.when(t > 0)
            def _():
                wait_dots(dots1_v, d1)

            compute(R, b0 + 1, dots1_v)

            @pl.when(t < nt - 1)
            def _():
                fire(b0 + 3, R, s1)

            pltpu.async_copy(dots1_v, out_hbm.at[base + b0 + 1], d1)
            return carry

        lax.fori_loop(0, nt, body, 0)
        wait_dots(dots0_v, d0)
        wait_dots(dots1_v, d1)

    return k(combined_idx, input_labels, in_embed, out_embed)


def _tc_loss(dots, B):
    bblk = 512

    def body(d_ref, o_ref):
        x = d_ref[...]
        col = lax.broadcasted_iota(jnp.int32, x.shape, 1)
        y = jnp.where(col < PP, x, -x)
        ls = jax.nn.log_sigmoid(y)
        ls = jnp.where(col < PP + NN, ls, 0.0)
        o_ref[...] = -jnp.sum(ls, axis=1)

    return pl.pallas_call(
        body,
        grid=(B // bblk,),
        in_specs=[pl.BlockSpec((bblk, R), lambda i: (i, 0))],
        out_specs=pl.BlockSpec((bblk,), lambda i: (i,)),
        out_shape=jax.ShapeDtypeStruct((B,), jnp.float32),
    )(dots)


def kernel(input_labels, pos_labels, neg_labels, in_embed, out_embed):
    B = input_labels.shape[0]
    pad = jnp.zeros((B, R - PP - NN), jnp.int32)
    combined = jnp.concatenate(
        [pos_labels, neg_labels, pad], axis=1).reshape(-1)
    dots = _sc_dots(combined, input_labels, in_embed, out_embed, B)
    return _tc_loss(dots, B)


# speedup vs baseline: 2.2986x; 1.0031x over previous
import functools

import jax
import jax.numpy as jnp
from jax import lax
from jax.experimental import pallas as pl
from jax.experimental.pallas import tpu as pltpu
from jax.experimental.pallas import tpu_sc as plsc

NC, NS = 2, 16
NW = NC * NS
D = 128
PP = 20
NN = 200
R = 224
RH = R // 2


def _sc_dots(combined_idx, input_labels, in_embed, out_embed, B):
    b_per_w = B // NW
    mesh = plsc.VectorSubcoreMesh(core_axis_name="c", subcore_axis_name="s")

    @functools.partial(
        pl.kernel,
        mesh=mesh,
        compiler_params=pltpu.CompilerParams(needs_layout_passes=False),
        out_type=jax.ShapeDtypeStruct((B, R), jnp.float32),
        scratch_types=[
            pltpu.VMEM((b_per_w,), jnp.int32),
            pltpu.VMEM((b_per_w, D), jnp.float32),
            pltpu.VMEM((b_per_w * R,), jnp.int32),
            pltpu.VMEM((2 * R, D), jnp.float32),
            pltpu.VMEM((R,), jnp.float32),
            pltpu.VMEM((R,), jnp.float32),
            pltpu.SemaphoreType.DMA,
            pltpu.SemaphoreType.DMA,
            pltpu.SemaphoreType.DMA,
            pltpu.SemaphoreType.DMA,
        ],
    )
    def k(idx_hbm, cidx_hbm, in_tab, out_tab, out_hbm,
          cidx_v, cent_v, idx_v, rows_v, dots0_v, dots1_v, s0, s1, d0, d1):
        wid = lax.axis_index("s") * NC + lax.axis_index("c")
        base = wid * b_per_w
        pltpu.sync_copy(cidx_hbm.at[pl.ds(base, b_per_w)], cidx_v)
        pltpu.sync_copy(idx_hbm.at[pl.ds(base * R, b_per_w * R)], idx_v)
        pltpu.async_copy(in_tab.at[cidx_v], cent_v, s0).wait()
        lane = lax.iota(jnp.int32, 16)

        def fire(b, row_off, sem):
            off = pl.multiple_of(b * R, 8)
            pltpu.async_copy(
                out_tab.at[idx_v.at[pl.ds(off, RH)]],
                rows_v.at[pl.ds(row_off, RH)], sem)
            pltpu.async_copy(
                out_tab.at[idx_v.at[pl.ds(off + RH, RH)]],
                rows_v.at[pl.ds(row_off + RH, RH)], sem)

        def wait_rows(sem, row_off):
            pltpu.make_async_copy(
                out_tab.at[pl.ds(0, R)],
                rows_v.at[pl.ds(row_off, R)], sem).wait()

        def wait_dots(dots_ref, sem):
            pltpu.make_async_copy(
                dots_ref, out_hbm.at[base], sem).wait()

        m15 = lane == 15

        def compute(row_base, bl, dots_ref):
            cs = [cent_v[bl, pl.ds(16 * j, 16)] for j in range(8)]

            def per16(g, c2):
                gbase = jnp.full((16,), g * 16, jnp.int32)
                for k in range(16):
                    row = row_base + g * 16 + k
                    p = [rows_v[row, pl.ds(16 * j, 16)] * cs[j]
                         for j in range(8)]
                    acc = (((p[0] + p[1]) + (p[2] + p[3]))
                           + ((p[4] + p[5]) + (p[6] + p[7])))
                    s = plsc.cumsum(acc)
                    plsc.store_scatter(dots_ref, [gbase + k], s, mask=m15)
                return c2

            lax.fori_loop(0, R // 16, per16, 0)

        fire(0, 0, s0)
        fire(1, R, s1)
        nt = b_per_w // 2

        def body(t, carry):
            b0 = 2 * t
            wait_rows(s0, 0)

            @pl.when(t > 0)
            def _():
                wait_dots(dots0_v, d0)

            compute(0, b0, dots0_v)

            @pl.when(t < nt - 1)
            def _():
                fire(b0 + 2, 0, s0)

            pltpu.async_copy(dots0_v, out_hbm.at[base + b0], d0)

            wait_rows(s1, R)

            @pl.when(t > 0)
            def _():
                wait_dots(dots1_v, d1)

            compute(R, b0 + 1, dots1_v)

            @pl.when(t < nt - 1)
            def _():
                fire(b0 + 3, R, s1)

            pltpu.async_copy(dots1_v, out_hbm.at[base + b0 + 1], d1)
            return carry

        lax.fori_loop(0, nt, body, 0)
        wait_dots(dots0_v, d0)
        wait_dots(dots1_v, d1)

    return k(combined_idx, input_labels, in_embed, out_embed)


def _tc_loss(dots, B):
    bblk = 512

    def body(d_ref, o_ref):
        x = d_ref[...]
        col = lax.broadcasted_iota(jnp.int32, x.shape, 1)
        y = jnp.where(col < PP, x, -x)
        ls = jax.nn.log_sigmoid(y)
        ls = jnp.where(col < PP + NN, ls, 0.0)
        o_ref[...] = -jnp.sum(ls, axis=1)

    return pl.pallas_call(
        body,
        grid=(B // bblk,),
        in_specs=[pl.BlockSpec((bblk, R), lambda i: (i, 0))],
        out_specs=pl.BlockSpec((bblk,), lambda i: (i,)),
        out_shape=jax.ShapeDtypeStruct((B,), jnp.float32),
    )(dots)


def kernel(input_labels, pos_labels, neg_labels, in_embed, out_embed):
    B = input_labels.shape[0]
    pad = jnp.zeros((B, R - PP - NN), jnp.int32)
    combined = jnp.concatenate(
        [pos_labels, neg_labels, pad], axis=1).reshape(-1)
    dots = _sc_dots(combined, input_labels, in_embed, out_embed, B)
    return _tc_loss(dots, B)
